# baseline (speedup 1.0000x reference)
import jax
import jax.numpy as jnp
from jax import lax
from jax.experimental import pallas as pl
from jax.experimental.pallas import tpu as pltpu
from jax.experimental.pallas import tpu_sc as plsc

N = 10000
E = 160000
IN_FEATS = 256
N_HIDDEN = 512
N_CLASSES = 64
ALPHA = 0.1
K_STEPS = 10

NC = 2
NS = 16
CW = N_CLASSES // NC
EPT = E // NS
CH = 125
NCH = EPT // CH
NPAD = 10240
RPT = NPAD // NS
TAIL = N - (NS - 1) * RPT

NBUF = 5
SLACK = 3
GDEPTH = NBUF - SLACK


def _mlp_body(x_ref, w1_ref, b1_ref, w2_ref, b2_ref, w3_ref, b3_ref, o_ref):
    bf = jnp.bfloat16
    h = jnp.maximum(
        jnp.dot(x_ref[...].astype(bf), w1_ref[...].astype(bf),
                preferred_element_type=jnp.float32) + b1_ref[...], 0.0)
    h = jnp.maximum(
        jnp.dot(h.astype(bf), w2_ref[...].astype(bf),
                preferred_element_type=jnp.float32) + b2_ref[...], 0.0)
    o_ref[...] = (
        jnp.dot(h.astype(bf), w3_ref[...].astype(bf),
                preferred_element_type=jnp.float32) + b3_ref[...])


def _mlp(features, W1, b1, W2, b2, W3, b3):
    blk = 1000
    full = lambda shape: pl.BlockSpec(shape, lambda i: (0, 0))
    return pl.pallas_call(
        _mlp_body,
        grid=(N // blk,),
        in_specs=[
            pl.BlockSpec((blk, IN_FEATS), lambda i: (i, 0)),
            full((IN_FEATS, N_HIDDEN)), full((1, N_HIDDEN)),
            full((N_HIDDEN, N_HIDDEN)), full((1, N_HIDDEN)),
            full((N_HIDDEN, N_CLASSES)), full((1, N_CLASSES)),
        ],
        out_specs=pl.BlockSpec((blk, N_CLASSES), lambda i: (i, 0)),
        out_shape=jax.ShapeDtypeStruct((N, N_CLASSES), jnp.float32),
    )(features, W1, b1.reshape(1, -1), W2, b2.reshape(1, -1),
      W3, b3.reshape(1, -1))


def _rsqrt16(x):
    y = plsc.bitcast(jnp.int32(0x5F3759DF)
                     - lax.shift_right_logical(plsc.bitcast(x, jnp.int32), 1),
                     jnp.float32)
    for _ in range(3):
        y = y * (1.5 - 0.5 * x * y * y)
    return y


def _prop_body(h_hbm, src_hbm, dst_hbm, zeros_hbm,
               out_hbm, work_hbm,
               src_v, dst_v, msg_v, s0res, u1res, upd_v,
               agg_sh, *sems):
    gsem = sems[:NBUF]
    ssem = sems[NBUF:2 * NBUF]
    c = lax.axis_index("c")
    s = lax.axis_index("s")
    rows = pl.ds(s * RPT, RPT)
    cols = pl.ds(c * CW, CW)
    mine = work_hbm.at[c]
    LO = pl.ds(0, 16)
    HI = pl.ds(16, 16)

    pltpu.sync_copy(src_hbm.at[s], src_v)
    pltpu.sync_copy(dst_hbm.at[s], dst_v)
    pltpu.sync_copy(zeros_hbm, agg_sh.at[rows])

    @pl.when(s < NS - 1)
    def _():
        pltpu.sync_copy(h_hbm.at[rows, cols], s0res)

    @pl.when(s == NS - 1)
    def _():
        pltpu.sync_copy(h_hbm.at[pl.ds((NS - 1) * RPT, TAIL), cols],
                        s0res.at[pl.ds(0, TAIL)])

    ones = msg_v.at[0]

    def fill_ones(r, _):
        for lanes in (LO, HI):
            ones[r, lanes] = jnp.full((16,), 1.0, jnp.float32)
        return 0

    lax.fori_loop(0, CH, fill_ones, 0, unroll=4)
    plsc.subcore_barrier()

    def deg_fire(j, _):
        pltpu.async_copy(ones, agg_sh.at[dst_v.at[j]], ssem[0], add=True)
        return 0

    def deg_drain(j, _):
        pltpu.make_async_copy(ones, agg_sh.at[dst_v.at[j]], ssem[0]).wait()
        return 0

    lax.fori_loop(0, NCH, deg_fire, 0)
    lax.fori_loop(0, NCH, deg_drain, 0)
    plsc.subcore_barrier()

    pltpu.sync_copy(agg_sh.at[rows], upd_v)
    pltpu.sync_copy(zeros_hbm, agg_sh.at[rows])

    def coeffs(r, _):
        d = jnp.maximum(upd_v[r, LO], 1.0)
        n = _rsqrt16(d)
        u1res[r, LO] = (1.0 - ALPHA) * n * n
        u1res[r, HI] = d * n
        for lanes in (LO, HI):
            s0 = n * s0res[r, lanes]
            upd_v[r, lanes] = s0
            s0res[r, lanes] = ALPHA * s0
        return 0

    lax.fori_loop(0, RPT, coeffs, 0, unroll=2)
    pltpu.sync_copy(upd_v, mine.at[rows])
    plsc.subcore_barrier()

    STEADY = NCH - GDEPTH - SLACK

    def _wait_gather(j, b):
        pltpu.make_async_copy(mine.at[src_v.at[j]], msg_v.at[b],
                              gsem[b]).wait()

    def _scatter(j, b):
        pltpu.async_copy(msg_v.at[b], agg_sh.at[dst_v.at[j]], ssem[b],
                         add=True)

    def _drain(j, b):
        pltpu.make_async_copy(msg_v.at[b], agg_sh.at[dst_v.at[j]],
                              ssem[b]).wait()

    def edge_pass():
        for b in range(GDEPTH):
            pltpu.async_copy(mine.at[src_v.at[b]], msg_v.at[b], gsem[b])
        for j in range(SLACK):
            pb = (j + NBUF - SLACK) % NBUF
            pltpu.async_copy(mine.at[src_v.at[j + GDEPTH]], msg_v.at[pb],
                             gsem[pb])
            _wait_gather(j, j % NBUF)
            _scatter(j, j % NBUF)

        def chunk(g, _):
            for b in range(NBUF):
                j = SLACK + g * NBUF + b
                jb = (SLACK + b) % NBUF
                pb = (jb + NBUF - SLACK) % NBUF
                _drain(j - SLACK, pb)
                pltpu.async_copy(mine.at[src_v.at[j + GDEPTH]], msg_v.at[pb],
                                 gsem[pb])
                _wait_gather(j, jb)
                _scatter(j, jb)
            return 0

        lax.fori_loop(0, STEADY // NBUF, chunk, 0)
        for j in range(NCH - GDEPTH, NCH):
            _drain(j - SLACK, (j - SLACK) % NBUF)
            _wait_gather(j, j % NBUF)
            _scatter(j, j % NBUF)
        for j in range(NCH - SLACK, NCH):
            _drain(j, j % NBUF)
        plsc.subcore_barrier()
        pltpu.sync_copy(agg_sh.at[rows], upd_v)
        pltpu.sync_copy(zeros_hbm, agg_sh.at[rows])

    def update(r, _):
        u1 = u1res[r, LO]
        for lanes in (LO, HI):
            upd_v[r, lanes] = u1 * upd_v[r, lanes] + s0res[r, lanes]
        return 0

    def step(t, _):
        edge_pass()
        lax.fori_loop(0, RPT, update, 0, unroll=4)
        pltpu.sync_copy(upd_v, mine.at[rows])
        plsc.subcore_barrier()
        return 0

    lax.fori_loop(0, K_STEPS - 1, step, 0)

    edge_pass()

    def finalize(r, _):
        u1 = u1res[r, LO]
        inv = u1res[r, HI]
        for lanes in (LO, HI):
            upd_v[r, lanes] = inv * (u1 * upd_v[r, lanes] + s0res[r, lanes])
        return 0

    lax.fori_loop(0, RPT, finalize, 0, unroll=4)

    @pl.when(s < NS - 1)
    def _():
        pltpu.sync_copy(upd_v, out_hbm.at[rows, cols])

    @pl.when(s == NS - 1)
    def _():
        pltpu.sync_copy(upd_v.at[pl.ds(0, TAIL)],
                        out_hbm.at[pl.ds((NS - 1) * RPT, TAIL), cols])


def _propagate(h, src2, dst2, zeros):
    mesh = plsc.VectorSubcoreMesh(core_axis_name="c", subcore_axis_name="s",
                                  num_cores=NC, num_subcores=NS)
    outs = pl.kernel(
        _prop_body,
        out_type=[
            jax.ShapeDtypeStruct((N, N_CLASSES), jnp.float32),
            jax.ShapeDtypeStruct((NC, NPAD, CW), jnp.float32),
        ],
        mesh=mesh,
        compiler_params=pltpu.CompilerParams(use_tc_tiling_on_sc=False,
                                             needs_layout_passes=False),
        scratch_types=[
            pltpu.VMEM((NCH, CH), jnp.int32),
            pltpu.VMEM((NCH, CH), jnp.int32),
            pltpu.VMEM((NBUF, CH, CW), jnp.float32),
            pltpu.VMEM((RPT, CW), jnp.float32),
            pltpu.VMEM((RPT, CW), jnp.float32),
            pltpu.VMEM((RPT, CW), jnp.float32),
            pltpu.VMEM_SHARED((NPAD, CW), jnp.float32),
        ] + [pltpu.SemaphoreType.DMA] * (2 * NBUF),
    )(h, src2, dst2, zeros)
    return outs[0]


def kernel(features, edge_index, W1, b1, W2, b2, W3, b3):
    src2 = edge_index[0].reshape(NS, NCH, CH)
    dst2 = edge_index[1].reshape(NS, NCH, CH)
    zeros_agg = jnp.zeros((RPT, CW), jnp.float32)

    h = _mlp(features, W1, b1, W2, b2, W3, b3)
    return _propagate(h, src2, dst2, zeros_agg)


# speedup vs baseline: 1.0657x; 1.0657x over previous
import jax
import jax.numpy as jnp
from jax import lax
from jax.experimental import pallas as pl
from jax.experimental.pallas import tpu as pltpu
from jax.experimental.pallas import tpu_sc as plsc

N = 10000
E = 160000
IN_FEATS = 256
N_HIDDEN = 512
N_CLASSES = 64
ALPHA = 0.1
K_STEPS = 10

NC = 2
NS = 16
CW = N_CLASSES // NC
EPT = E // NS
CH = 125
NCH = EPT // CH
NPAD = 10240
RPT = NPAD // NS
TAIL = N - (NS - 1) * RPT

NBUF = 5
SLACK = 2
GDEPTH = NBUF - SLACK


def _mlp_body(x_ref, w1_ref, b1_ref, w2_ref, b2_ref, w3_ref, b3_ref, o_ref):
    bf = jnp.bfloat16
    h = jnp.maximum(
        jnp.dot(x_ref[...].astype(bf), w1_ref[...].astype(bf),
                preferred_element_type=jnp.float32) + b1_ref[...], 0.0)
    h = jnp.maximum(
        jnp.dot(h.astype(bf), w2_ref[...].astype(bf),
                preferred_element_type=jnp.float32) + b2_ref[...], 0.0)
    o_ref[...] = (
        jnp.dot(h.astype(bf), w3_ref[...].astype(bf),
                preferred_element_type=jnp.float32) + b3_ref[...])


def _mlp(features, W1, b1, W2, b2, W3, b3):
    blk = 1000
    full = lambda shape: pl.BlockSpec(shape, lambda i: (0, 0))
    return pl.pallas_call(
        _mlp_body,
        grid=(N // blk,),
        in_specs=[
            pl.BlockSpec((blk, IN_FEATS), lambda i: (i, 0)),
            full((IN_FEATS, N_HIDDEN)), full((1, N_HIDDEN)),
            full((N_HIDDEN, N_HIDDEN)), full((1, N_HIDDEN)),
            full((N_HIDDEN, N_CLASSES)), full((1, N_CLASSES)),
        ],
        out_specs=pl.BlockSpec((blk, N_CLASSES), lambda i: (i, 0)),
        out_shape=jax.ShapeDtypeStruct((N, N_CLASSES), jnp.float32),
    )(features, W1, b1.reshape(1, -1), W2, b2.reshape(1, -1),
      W3, b3.reshape(1, -1))


def _rsqrt16(x):
    y = plsc.bitcast(jnp.int32(0x5F3759DF)
                     - lax.shift_right_logical(plsc.bitcast(x, jnp.int32), 1),
                     jnp.float32)
    for _ in range(3):
        y = y * (1.5 - 0.5 * x * y * y)
    return y


def _prop_body(h_hbm, src_hbm, dst_hbm, zeros_hbm,
               out_hbm, work_hbm,
               src_v, dst_v, msg_v, s0res, u1res, upd_v,
               agg_sh, *sems):
    gsem = sems[:NBUF]
    ssem = sems[NBUF:2 * NBUF]
    c = lax.axis_index("c")
    s = lax.axis_index("s")
    rows = pl.ds(s * RPT, RPT)
    cols = pl.ds(c * CW, CW)
    mine = work_hbm.at[c]
    LO = pl.ds(0, 16)
    HI = pl.ds(16, 16)

    pltpu.sync_copy(src_hbm.at[s], src_v)
    pltpu.sync_copy(dst_hbm.at[s], dst_v)
    pltpu.sync_copy(zeros_hbm, agg_sh.at[rows])

    @pl.when(s < NS - 1)
    def _():
        pltpu.sync_copy(h_hbm.at[rows, cols], s0res)

    @pl.when(s == NS - 1)
    def _():
        pltpu.sync_copy(h_hbm.at[pl.ds((NS - 1) * RPT, TAIL), cols],
                        s0res.at[pl.ds(0, TAIL)])

    ones = msg_v.at[0]

    def fill_ones(r, _):
        for lanes in (LO, HI):
            ones[r, lanes] = jnp.full((16,), 1.0, jnp.float32)
        return 0

    lax.fori_loop(0, CH, fill_ones, 0, unroll=4)
    plsc.subcore_barrier()

    def deg_fire(j, _):
        pltpu.async_copy(ones, agg_sh.at[dst_v.at[j]], ssem[0], add=True)
        return 0

    def deg_drain(j, _):
        pltpu.make_async_copy(ones, agg_sh.at[dst_v.at[j]], ssem[0]).wait()
        return 0

    lax.fori_loop(0, NCH, deg_fire, 0)
    lax.fori_loop(0, NCH, deg_drain, 0)
    plsc.subcore_barrier()

    pltpu.sync_copy(agg_sh.at[rows], upd_v)
    pltpu.sync_copy(zeros_hbm, agg_sh.at[rows])

    def coeffs(r, _):
        d = jnp.maximum(upd_v[r, LO], 1.0)
        n = _rsqrt16(d)
        u1res[r, LO] = (1.0 - ALPHA) * n * n
        u1res[r, HI] = d * n
        for lanes in (LO, HI):
            s0 = n * s0res[r, lanes]
            upd_v[r, lanes] = s0
            s0res[r, lanes] = ALPHA * s0
        return 0

    lax.fori_loop(0, RPT, coeffs, 0, unroll=2)
    pltpu.sync_copy(upd_v, mine.at[rows])
    plsc.subcore_barrier()

    STEADY = NCH - GDEPTH - SLACK

    def _wait_gather(j, b):
        pltpu.make_async_copy(mine.at[src_v.at[j]], msg_v.at[b],
                              gsem[b]).wait()

    def _scatter(j, b):
        pltpu.async_copy(msg_v.at[b], agg_sh.at[dst_v.at[j]], ssem[b],
                         add=True)

    def _drain(j, b):
        pltpu.make_async_copy(msg_v.at[b], agg_sh.at[dst_v.at[j]],
                              ssem[b]).wait()

    def edge_pass():
        for b in range(GDEPTH):
            pltpu.async_copy(mine.at[src_v.at[b]], msg_v.at[b], gsem[b])
        for j in range(SLACK):
            pb = (j + NBUF - SLACK) % NBUF
            pltpu.async_copy(mine.at[src_v.at[j + GDEPTH]], msg_v.at[pb],
                             gsem[pb])
            _wait_gather(j, j % NBUF)
            _scatter(j, j % NBUF)

        def chunk(g, _):
            for b in range(NBUF):
                j = SLACK + g * NBUF + b
                jb = (SLACK + b) % NBUF
                pb = (jb + NBUF - SLACK) % NBUF
                _drain(j - SLACK, pb)
                pltpu.async_copy(mine.at[src_v.at[j + GDEPTH]], msg_v.at[pb],
                                 gsem[pb])
                _wait_gather(j, jb)
                _scatter(j, jb)
            return 0

        lax.fori_loop(0, STEADY // NBUF, chunk, 0)
        for j in range(NCH - GDEPTH, NCH):
            _drain(j - SLACK, (j - SLACK) % NBUF)
            _wait_gather(j, j % NBUF)
            _scatter(j, j % NBUF)
        for j in range(NCH - SLACK, NCH):
            _drain(j, j % NBUF)
        plsc.subcore_barrier()
        pltpu.sync_copy(agg_sh.at[rows], upd_v)
        pltpu.sync_copy(zeros_hbm, agg_sh.at[rows])

    def update(r, _):
        u1 = u1res[r, LO]
        for lanes in (LO, HI):
            upd_v[r, lanes] = u1 * upd_v[r, lanes] + s0res[r, lanes]
        return 0

    def step(t, _):
        edge_pass()
        lax.fori_loop(0, RPT, update, 0, unroll=4)
        pltpu.sync_copy(upd_v, mine.at[rows])
        plsc.subcore_barrier()
        return 0

    lax.fori_loop(0, K_STEPS - 1, step, 0)

    edge_pass()

    def finalize(r, _):
        u1 = u1res[r, LO]
        inv = u1res[r, HI]
        for lanes in (LO, HI):
            upd_v[r, lanes] = inv * (u1 * upd_v[r, lanes] + s0res[r, lanes])
        return 0

    lax.fori_loop(0, RPT, finalize, 0, unroll=4)

    @pl.when(s < NS - 1)
    def _():
        pltpu.sync_copy(upd_v, out_hbm.at[rows, cols])

    @pl.when(s == NS - 1)
    def _():
        pltpu.sync_copy(upd_v.at[pl.ds(0, TAIL)],
                        out_hbm.at[pl.ds((NS - 1) * RPT, TAIL), cols])


def _propagate(h, src2, dst2, zeros):
    mesh = plsc.VectorSubcoreMesh(core_axis_name="c", subcore_axis_name="s",
                                  num_cores=NC, num_subcores=NS)
    outs = pl.kernel(
        _prop_body,
        out_type=[
            jax.ShapeDtypeStruct((N, N_CLASSES), jnp.float32),
            jax.ShapeDtypeStruct((NC, NPAD, CW), jnp.float32),
        ],
        mesh=mesh,
        compiler_params=pltpu.CompilerParams(use_tc_tiling_on_sc=False,
                                             needs_layout_passes=False),
        scratch_types=[
            pltpu.VMEM((NCH, CH), jnp.int32),
            pltpu.VMEM((NCH, CH), jnp.int32),
            pltpu.VMEM((NBUF, CH, CW), jnp.float32),
            pltpu.VMEM((RPT, CW), jnp.float32),
            pltpu.VMEM((RPT, CW), jnp.float32),
            pltpu.VMEM((RPT, CW), jnp.float32),
            pltpu.VMEM_SHARED((NPAD, CW), jnp.float32),
        ] + [pltpu.SemaphoreType.DMA] * (2 * NBUF),
    )(h, src2, dst2, zeros)
    return outs[0]


def kernel(features, edge_index, W1, b1, W2, b2, W3, b3):
    src2 = edge_index[0].reshape(NS, NCH, CH)
    dst2 = edge_index[1].reshape(NS, NCH, CH)
    zeros_agg = jnp.zeros((RPT, CW), jnp.float32)

    h = _mlp(features, W1, b1, W2, b2, W3, b3)
    return _propagate(h, src2, dst2, zeros_agg)
